# Initial kernel scaffold; baseline (speedup 1.0000x reference)
import jax
import jax.numpy as jnp
from jax import lax
from jax.experimental import pallas as pl
from jax.experimental.pallas import tpu as pltpu
from jax.experimental.pallas import tpu_sc as plsc

N = 10000
E = 320000
D = 128
N_PAD = 10240
NC = 2
NS = 16
NW = NC * NS
EPT = E // NW
RPT = N_PAD // NS
NPT = N_PAD // NW
BLK = 1000
GB = 5


def _proj_body(x_ref, w_ref, b_ref, a1_ref, a2_ref, ab_ref,
               h_ref, s1_ref, s2_ref, c_ref, mx_ref):
    i = pl.program_id(0)
    x = x_ref[...]
    h = lax.dot_general(x, w_ref[...], (((1,), (1,)), ((), ())),
                        preferred_element_type=jnp.float32) + b_ref[...]
    h_ref[...] = h
    s1 = lax.dot_general(h, a1_ref[...], (((1,), (1,)), ((), ())),
                         preferred_element_type=jnp.float32) + ab_ref[0]
    s2 = lax.dot_general(h, a2_ref[...], (((1,), (1,)), ((), ())),
                         preferred_element_type=jnp.float32)
    s1_ref[...] = s1
    s2_ref[...] = s2

    @pl.when(i == 0)
    def _():
        mx_ref[0] = jnp.float32(-1e30)
        mx_ref[1] = jnp.float32(-1e30)

    mx_ref[0] = jnp.maximum(mx_ref[0], jnp.max(s1))
    mx_ref[1] = jnp.maximum(mx_ref[1], jnp.max(s2))

    @pl.when(i == (N // BLK) - 1)
    def _():
        t = mx_ref[0] + mx_ref[1]
        c_ref[0] = jnp.where(t >= 0.0, t, 0.01 * t)


def _project(x, W, b_w, a1, a2, a_b):
    return pl.pallas_call(
        _proj_body,
        grid=(N // BLK,),
        in_specs=[
            pl.BlockSpec((BLK, D), lambda i: (i, 0)),
            pl.BlockSpec((D, D), lambda i: (0, 0)),
            pl.BlockSpec((1, D), lambda i: (0, 0)),
            pl.BlockSpec((1, D), lambda i: (0, 0)),
            pl.BlockSpec((1, D), lambda i: (0, 0)),
            pl.BlockSpec(memory_space=pltpu.SMEM),
        ],
        out_specs=[
            pl.BlockSpec((BLK, D), lambda i: (i, 0)),
            pl.BlockSpec((BLK, 1), lambda i: (i, 0)),
            pl.BlockSpec((BLK, 1), lambda i: (i, 0)),
            pl.BlockSpec(memory_space=pltpu.SMEM),
        ],
        out_shape=[
            jax.ShapeDtypeStruct((N, D), jnp.float32),
            jax.ShapeDtypeStruct((N, 1), jnp.float32),
            jax.ShapeDtypeStruct((N, 1), jnp.float32),
            jax.ShapeDtypeStruct((1,), jnp.float32),
        ],
        scratch_shapes=[pltpu.SMEM((2,), jnp.float32)],
    )(x, W, b_w, a1, a2, a_b)


def _edge_sweep(src, dst, s1, s2, c16, h):
    mesh = plsc.VectorSubcoreMesh(core_axis_name="c", subcore_axis_name="s",
                                  num_cores=NC, num_subcores=NS)

    def body(src_hbm, dst_hbm, s1_hbm, s2_hbm, c_hbm, h_hbm,
             numer_hbm, den_hbm,
             src_v, dst_v, s1_v, s2_v, den_v, c_v, zbuf_v, rows_v, acc_sh,
             sem):
        cid = lax.axis_index("c")
        sid = lax.axis_index("s")
        wid = cid * NS + sid
        ebase = wid * EPT

        pltpu.sync_copy(src_hbm.at[pl.ds(ebase, EPT)], src_v)
        pltpu.sync_copy(dst_hbm.at[pl.ds(ebase, EPT)], dst_v)
        pltpu.sync_copy(s1_hbm, s1_v)
        pltpu.sync_copy(s2_hbm, s2_v)
        pltpu.sync_copy(c_hbm, c_v)
        cvec = c_v[...]
        z16 = jnp.zeros((16,), jnp.float32)

        def zb(i, carry):
            for j in range(8):
                zbuf_v[i, pl.ds(j * 16, 16)] = z16
            return carry

        lax.fori_loop(0, 128, zb, 0)

        def zd(i, carry):
            den_v[pl.ds(i * 16, 16)] = z16
            return carry

        lax.fori_loop(0, N_PAD // 16, zd, 0)

        rbase = sid * RPT

        def za(i, carry):
            pltpu.sync_copy(zbuf_v, acc_sh.at[pl.ds(rbase + i * 128, 128), :])
            return carry

        lax.fori_loop(0, RPT // 128, za, 0)
        plsc.subcore_barrier()

        def group(g, carry):
            off = g * (GB * 16)
            for b in range(GB):
                isrc = src_v[pl.ds(off + b * 16, 16)]
                pltpu.async_copy(h_hbm.at[isrc], rows_v.at[b], sem)
            for b in range(GB):
                isrc = src_v[pl.ds(off + b * 16, 16)]
                pltpu.make_async_copy(h_hbm.at[isrc], rows_v.at[b], sem).wait()
            for b in range(GB):
                o = off + b * 16
                isrc = src_v[pl.ds(o, 16)]
                idst = dst_v[pl.ds(o, 16)]
                sd = plsc.load_gather(s1_v, [idst])
                ss = plsc.load_gather(s2_v, [isrc])
                t = sd + ss
                e = jnp.where(t >= 0.0, t, 0.01 * t)
                p = jnp.exp(e - cvec)
                plsc.addupdate_scatter(den_v, [idst], p)
                for r in range(16):
                    pr = jnp.broadcast_to(p[r], (16,))
                    for j in range(8):
                        sl = pl.ds(j * 16, 16)
                        rows_v[b, r, sl] = rows_v[b, r, sl] * pr
                pltpu.sync_copy(rows_v.at[b], acc_sh.at[idst], add=True)
            return carry

        lax.fori_loop(0, EPT // (GB * 16), group, 0)

        plsc.subcore_barrier()
        pltpu.sync_copy(acc_sh.at[pl.ds(rbase, RPT), :],
                        numer_hbm.at[cid, pl.ds(rbase, RPT), :])
        pltpu.sync_copy(den_v, den_hbm.at[wid])

    run = pl.kernel(
        body,
        out_type=[
            jax.ShapeDtypeStruct((NC, N_PAD, D), jnp.float32),
            jax.ShapeDtypeStruct((NW, N_PAD), jnp.float32),
        ],
        mesh=mesh,
        scratch_types=[
            pltpu.VMEM((EPT,), jnp.int32),
            pltpu.VMEM((EPT,), jnp.int32),
            pltpu.VMEM((N,), jnp.float32),
            pltpu.VMEM((N,), jnp.float32),
            pltpu.VMEM((N_PAD,), jnp.float32),
            pltpu.VMEM((16,), jnp.float32),
            pltpu.VMEM((128, D), jnp.float32),
            pltpu.VMEM((GB, 16, D), jnp.float32),
            pltpu.VMEM_SHARED((N_PAD, D), jnp.float32),
            pltpu.SemaphoreType.DMA,
        ],
    )
    return run(src, dst, s1, s2, c16, h)


def _sc_fin_body(numer_hbm, den_hbm, out_hbm, a_v, b_v, d_v, ds_v):
    cid = lax.axis_index("c")
    sid = lax.axis_index("s")
    wid = cid * NS + sid
    base = wid * NPT

    pltpu.sync_copy(numer_hbm.at[0, pl.ds(base, NPT), :], a_v)
    pltpu.sync_copy(numer_hbm.at[1, pl.ds(base, NPT), :], b_v)
    pltpu.sync_copy(den_hbm.at[:, pl.ds(base, NPT)], d_v)

    def dred(i, carry):
        sl = pl.ds(i * 16, 16)
        acc = d_v[0, sl]
        for r in range(1, NW):
            acc = acc + d_v[r, sl]
        ds_v[sl] = acc
        return carry

    lax.fori_loop(0, NPT // 16, dred, 0)

    def rowdiv(r, carry):
        inv = 1.0 / (ds_v[r] + 1e-16)
        iv = jnp.broadcast_to(inv, (16,))
        for j in range(8):
            sl = pl.ds(j * 16, 16)
            a_v[r, sl] = (a_v[r, sl] + b_v[r, sl]) * iv
        return carry

    lax.fori_loop(0, NPT, rowdiv, 0)
    pltpu.sync_copy(a_v, out_hbm.at[pl.ds(base, NPT), :])


def _finalize(numer, denom):
    mesh = plsc.VectorSubcoreMesh(core_axis_name="c", subcore_axis_name="s",
                                  num_cores=NC, num_subcores=NS)
    run = pl.kernel(
        _sc_fin_body,
        out_type=jax.ShapeDtypeStruct((N_PAD, D), jnp.float32),
        mesh=mesh,
        scratch_types=[
            pltpu.VMEM((NPT, D), jnp.float32),
            pltpu.VMEM((NPT, D), jnp.float32),
            pltpu.VMEM((NW, NPT), jnp.float32),
            pltpu.VMEM((NPT,), jnp.float32),
        ],
    )
    return run(numer, denom)


def kernel(x, edge_index, W, b_w, a_w, a_b):
    a1 = a_w[:, :D]
    a2 = a_w[:, D:]
    h, s1, s2, c = _project(x, W, b_w.reshape(1, D), a1, a2, a_b)
    src = edge_index[0]
    dst = edge_index[1]
    c16 = jnp.broadcast_to(c, (16,))
    numer, denom = _edge_sweep(src, dst, s1.reshape(N), s2.reshape(N), c16, h)
    out = _finalize(numer, denom)
    return out[:N]


# speedup vs baseline: 19.8806x; 19.8806x over previous
import jax
import jax.numpy as jnp
from jax import lax
from jax.experimental import pallas as pl
from jax.experimental.pallas import tpu as pltpu
from jax.experimental.pallas import tpu_sc as plsc

N = 10000
E = 320000
D = 128
N_PAD = 10240
NC = 2
NS = 16
NW = NC * NS
EPT = E // NW
RPT = N_PAD // NS
NPT = N_PAD // NW
BLK = 1000
GB = 5
IDXCAP = 2000


def _proj_body(x_ref, w_ref, b_ref, a1_ref, a2_ref, ab_ref,
               h_ref, s1_ref, s2_ref, c_ref, mx_ref):
    i = pl.program_id(0)
    x = x_ref[...]
    h = lax.dot_general(x, w_ref[...], (((1,), (1,)), ((), ())),
                        preferred_element_type=jnp.float32) + b_ref[...]
    h_ref[...] = h
    s1 = jnp.sum(h * a1_ref[...], axis=1, keepdims=True) + ab_ref[0]
    s2 = jnp.sum(h * a2_ref[...], axis=1, keepdims=True)
    s1_ref[...] = s1
    s2_ref[...] = s2

    @pl.when(i == 0)
    def _():
        mx_ref[0] = jnp.float32(-1e30)
        mx_ref[1] = jnp.float32(-1e30)

    mx_ref[0] = jnp.maximum(mx_ref[0], jnp.max(s1))
    mx_ref[1] = jnp.maximum(mx_ref[1], jnp.max(s2))

    @pl.when(i == (N // BLK) - 1)
    def _():
        t = mx_ref[0] + mx_ref[1]
        c_ref[0] = jnp.where(t >= 0.0, t, 0.01 * t)


def _project(x, W, b_w, a1, a2, a_b):
    return pl.pallas_call(
        _proj_body,
        grid=(N // BLK,),
        in_specs=[
            pl.BlockSpec((BLK, D), lambda i: (i, 0)),
            pl.BlockSpec((D, D), lambda i: (0, 0)),
            pl.BlockSpec((1, D), lambda i: (0, 0)),
            pl.BlockSpec((1, D), lambda i: (0, 0)),
            pl.BlockSpec((1, D), lambda i: (0, 0)),
            pl.BlockSpec(memory_space=pltpu.SMEM),
        ],
        out_specs=[
            pl.BlockSpec((BLK, D), lambda i: (i, 0)),
            pl.BlockSpec((BLK, 1), lambda i: (i, 0)),
            pl.BlockSpec((BLK, 1), lambda i: (i, 0)),
            pl.BlockSpec(memory_space=pltpu.SMEM),
        ],
        out_shape=[
            jax.ShapeDtypeStruct((N, D), jnp.float32),
            jax.ShapeDtypeStruct((N, 1), jnp.float32),
            jax.ShapeDtypeStruct((N, 1), jnp.float32),
            jax.ShapeDtypeStruct((1,), jnp.float32),
        ],
        scratch_shapes=[pltpu.SMEM((2,), jnp.float32)],
    )(x, W, b_w, a1, a2, a_b)


def _edge_sweep(src, dst, s1, s2, c16, h):
    mesh = plsc.VectorSubcoreMesh(core_axis_name="c", subcore_axis_name="s",
                                  num_cores=NC, num_subcores=NS)

    def body(src_hbm, dst_hbm, s1_hbm, s2_hbm, c_hbm, h_hbm,
             numer_hbm, den_hbm,
             src_v, dst_v, s1_v, s2_v, den_v, c_v, rows_v, acc_sh,
             sem):
        cid = lax.axis_index("c")
        sid = lax.axis_index("s")
        wid = cid * NS + sid
        ebase = wid * EPT

        pltpu.sync_copy(s1_hbm, s1_v)
        pltpu.sync_copy(s2_hbm, s2_v)
        pltpu.sync_copy(c_hbm, c_v)
        cvec = c_v[...]
        z16 = jnp.zeros((16,), jnp.float32)

        def zrows(i, carry):
            for j in range(8):
                rows_v[i, pl.ds(j * 16, 16)] = z16
            return carry

        lax.fori_loop(0, GB * 16, zrows, 0)

        def zd(i, carry):
            den_v[pl.ds(i * 16, 16)] = z16
            return carry

        lax.fori_loop(0, N_PAD // 16, zd, 0)

        rbase = sid * RPT

        def za(i, carry):
            pltpu.sync_copy(
                rows_v, acc_sh.at[pl.ds(rbase + i * (GB * 16), GB * 16), :])
            return carry

        lax.fori_loop(0, RPT // (GB * 16), za, 0)
        plsc.subcore_barrier()

        def refill(rr, carry):
            pltpu.sync_copy(src_hbm.at[pl.ds(ebase + rr * IDXCAP, IDXCAP)],
                            src_v)
            pltpu.sync_copy(dst_hbm.at[pl.ds(ebase + rr * IDXCAP, IDXCAP)],
                            dst_v)

            def group(g, carry2):
                off = g * (GB * 16)
                for b in range(GB):
                    isrc = src_v[pl.ds(off + b * 16, 16)]
                    pltpu.async_copy(h_hbm.at[isrc],
                                     rows_v.at[pl.ds(b * 16, 16)], sem)
                for b in range(GB):
                    isrc = src_v[pl.ds(off + b * 16, 16)]
                    pltpu.make_async_copy(h_hbm.at[isrc],
                                          rows_v.at[pl.ds(b * 16, 16)],
                                          sem).wait()
                for b in range(GB):
                    o = off + b * 16
                    isrc = src_v[pl.ds(o, 16)]
                    idst = dst_v[pl.ds(o, 16)]
                    sd = plsc.load_gather(s1_v, [idst])
                    ss = plsc.load_gather(s2_v, [isrc])
                    t = sd + ss
                    e = jnp.where(t >= 0.0, t, 0.01 * t)
                    p = jnp.exp(e - cvec)
                    plsc.addupdate_scatter(den_v, [idst], p)
                    for r in range(16):
                        br = b * 16 + r
                        pr = jnp.broadcast_to(p[r], (16,))
                        for j in range(8):
                            sl = pl.ds(j * 16, 16)
                            rows_v[br, sl] = rows_v[br, sl] * pr
                    pltpu.sync_copy(rows_v.at[pl.ds(b * 16, 16)],
                                    acc_sh.at[idst], add=True)
                return carry2

            lax.fori_loop(0, IDXCAP // (GB * 16), group, 0)
            return carry

        lax.fori_loop(0, EPT // IDXCAP, refill, 0)

        plsc.subcore_barrier()
        pltpu.sync_copy(acc_sh.at[pl.ds(rbase, RPT), :],
                        numer_hbm.at[cid, pl.ds(rbase, RPT), :])
        pltpu.sync_copy(den_v, den_hbm.at[wid])

    run = pl.kernel(
        body,
        out_type=[
            jax.ShapeDtypeStruct((NC, N_PAD, D), jnp.float32),
            jax.ShapeDtypeStruct((NW, N_PAD), jnp.float32),
        ],
        mesh=mesh,
        scratch_types=[
            pltpu.VMEM((IDXCAP,), jnp.int32),
            pltpu.VMEM((IDXCAP,), jnp.int32),
            pltpu.VMEM((N,), jnp.float32),
            pltpu.VMEM((N,), jnp.float32),
            pltpu.VMEM((N_PAD,), jnp.float32),
            pltpu.VMEM((16,), jnp.float32),
            pltpu.VMEM((GB * 16, D), jnp.float32),
            pltpu.VMEM_SHARED((N_PAD, D), jnp.float32),
            pltpu.SemaphoreType.DMA,
        ],
        compiler_params=pltpu.CompilerParams(needs_layout_passes=False),
    )
    return run(src, dst, s1, s2, c16, h)


def _sc_fin_body(numer_hbm, den_hbm, out_hbm, a_v, b_v, d_v, ds_v):
    cid = lax.axis_index("c")
    sid = lax.axis_index("s")
    wid = cid * NS + sid
    base = wid * NPT

    pltpu.sync_copy(numer_hbm.at[0, pl.ds(base, NPT), :], a_v)
    pltpu.sync_copy(numer_hbm.at[1, pl.ds(base, NPT), :], b_v)
    for r in range(NW):
        pltpu.sync_copy(den_hbm.at[pl.ds(r * N_PAD + base, NPT)],
                        d_v.at[pl.ds(r * NPT, NPT)])

    def dred(i, carry):
        sl = pl.ds(i * 16, 16)
        acc = d_v[pl.ds(i * 16, 16)]
        for r in range(1, NW):
            acc = acc + d_v[pl.ds(r * NPT + i * 16, 16)]
        ds_v[sl] = acc
        return carry

    lax.fori_loop(0, NPT // 16, dred, 0)

    def rowdiv(i, carry):
        inv = 1.0 / (ds_v[pl.ds(i * 16, 16)] + 1e-16)
        for rr in range(16):
            r = i * 16 + rr
            iv = jnp.broadcast_to(inv[rr], (16,))
            for j in range(8):
                sl = pl.ds(j * 16, 16)
                a_v[r, sl] = (a_v[r, sl] + b_v[r, sl]) * iv
        return carry

    lax.fori_loop(0, NPT // 16, rowdiv, 0)
    pltpu.sync_copy(a_v, out_hbm.at[pl.ds(base, NPT), :])


def _finalize(numer, denom):
    mesh = plsc.VectorSubcoreMesh(core_axis_name="c", subcore_axis_name="s",
                                  num_cores=NC, num_subcores=NS)
    run = pl.kernel(
        _sc_fin_body,
        out_type=jax.ShapeDtypeStruct((N_PAD, D), jnp.float32),
        mesh=mesh,
        scratch_types=[
            pltpu.VMEM((NPT, D), jnp.float32),
            pltpu.VMEM((NPT, D), jnp.float32),
            pltpu.VMEM((NW * NPT,), jnp.float32),
            pltpu.VMEM((NPT,), jnp.float32),
        ],
    )
    return run(numer, denom)


def kernel(x, edge_index, W, b_w, a_w, a_b):
    a1 = a_w[:, :D]
    a2 = a_w[:, D:]
    h, s1, s2, c = _project(x, W, b_w.reshape(1, D), a1, a2, a_b)
    src = edge_index[0]
    dst = edge_index[1]
    c16 = jnp.broadcast_to(c, (16,))
    numer, denom = _edge_sweep(src, dst, s1.reshape(N), s2.reshape(N), c16, h)
    out = _finalize(numer, denom.reshape(NW * N_PAD))
    return out[:N]


# speedup vs baseline: 36.3011x; 1.8260x over previous
import jax
import jax.numpy as jnp
from jax import lax
from jax.experimental import pallas as pl
from jax.experimental.pallas import tpu as pltpu
from jax.experimental.pallas import tpu_sc as plsc

N = 10000
E = 320000
D = 128
N_PAD = 10240
NC = 2
NS = 16
NW = NC * NS
EPT = E // NW
RPT = N_PAD // NS
NPT = N_PAD // NW
BLK = 1000
GB = 5
SB = 2
IDXCAP = 2000


def _proj_body(x_ref, w_ref, b_ref, a1_ref, a2_ref, ab_ref,
               h_ref, s1_ref, s2_ref, c_ref, mx_ref):
    i = pl.program_id(0)
    x = x_ref[...]
    h = lax.dot_general(x, w_ref[...], (((1,), (1,)), ((), ())),
                        preferred_element_type=jnp.float32) + b_ref[...]
    h_ref[...] = h
    s1 = jnp.sum(h * a1_ref[...], axis=1, keepdims=True) + ab_ref[0]
    s2 = jnp.sum(h * a2_ref[...], axis=1, keepdims=True)
    s1_ref[...] = s1
    s2_ref[...] = s2

    @pl.when(i == 0)
    def _():
        mx_ref[0] = jnp.float32(-1e30)
        mx_ref[1] = jnp.float32(-1e30)

    mx_ref[0] = jnp.maximum(mx_ref[0], jnp.max(s1))
    mx_ref[1] = jnp.maximum(mx_ref[1], jnp.max(s2))

    @pl.when(i == (N // BLK) - 1)
    def _():
        t = mx_ref[0] + mx_ref[1]
        c_ref[0] = jnp.where(t >= 0.0, t, 0.01 * t)


def _project(x, W, b_w, a1, a2, a_b):
    return pl.pallas_call(
        _proj_body,
        grid=(N // BLK,),
        in_specs=[
            pl.BlockSpec((BLK, D), lambda i: (i, 0)),
            pl.BlockSpec((D, D), lambda i: (0, 0)),
            pl.BlockSpec((1, D), lambda i: (0, 0)),
            pl.BlockSpec((1, D), lambda i: (0, 0)),
            pl.BlockSpec((1, D), lambda i: (0, 0)),
            pl.BlockSpec(memory_space=pltpu.SMEM),
        ],
        out_specs=[
            pl.BlockSpec((BLK, D), lambda i: (i, 0)),
            pl.BlockSpec((BLK, 1), lambda i: (i, 0)),
            pl.BlockSpec((BLK, 1), lambda i: (i, 0)),
            pl.BlockSpec(memory_space=pltpu.SMEM),
        ],
        out_shape=[
            jax.ShapeDtypeStruct((N, D), jnp.float32),
            jax.ShapeDtypeStruct((N, 1), jnp.float32),
            jax.ShapeDtypeStruct((N, 1), jnp.float32),
            jax.ShapeDtypeStruct((1,), jnp.float32),
        ],
        scratch_shapes=[pltpu.SMEM((2,), jnp.float32)],
    )(x, W, b_w, a1, a2, a_b)


def _edge_sweep(src, dst, s1, s2, c16, h):
    mesh = plsc.VectorSubcoreMesh(core_axis_name="c", subcore_axis_name="s",
                                  num_cores=NC, num_subcores=NS)

    RG = IDXCAP // (GB * 16)

    def body(src_hbm, dst_hbm, s1_hbm, s2_hbm, c_hbm, h_hbm,
             numer_hbm, den_hbm,
             src_v, dst_v, s1_v, s2_v, den_v, c_v, grows_v, srows_v, acc_sh,
             g_sems, s_sems, set_sem):
        cid = lax.axis_index("c")
        sid = lax.axis_index("s")
        wid = cid * NS + sid
        ebase = wid * EPT
        rbase = sid * RPT

        pltpu.async_copy(s1_hbm, s1_v, set_sem)
        pltpu.async_copy(s2_hbm, s2_v, set_sem)
        pltpu.async_copy(c_hbm, c_v, set_sem)
        z16 = jnp.zeros((16,), jnp.float32)

        def zg(i, carry):
            for j in range(8):
                grows_v[i, pl.ds(j * 16, 16)] = z16
            return carry

        lax.fori_loop(0, GB * 16, zg, 0)

        def zs(i, carry):
            for j in range(8):
                srows_v[i, pl.ds(j * 16, 16)] = z16
            return carry

        lax.fori_loop(0, SB * 16, zs, 0)

        def zd(i, carry):
            den_v[pl.ds(i * 16, 16)] = z16
            return carry

        lax.fori_loop(0, N_PAD // 16, zd, 0)

        def za(i, carry):
            pltpu.async_copy(
                grows_v, acc_sh.at[pl.ds(rbase + i * (GB * 16), GB * 16), :],
                set_sem)
            return carry

        lax.fori_loop(0, RPT // (GB * 16), za, 0)

        pltpu.make_async_copy(s1_hbm, s1_v, set_sem).wait()
        pltpu.make_async_copy(s2_hbm, s2_v, set_sem).wait()
        pltpu.make_async_copy(c_hbm, c_v, set_sem).wait()

        def zaw(i, carry):
            pltpu.make_async_copy(
                grows_v, acc_sh.at[pl.ds(rbase + i * (GB * 16), GB * 16), :],
                set_sem).wait()
            return carry

        lax.fori_loop(0, RPT // (GB * 16), zaw, 0)
        plsc.subcore_barrier()
        cvec = c_v[...]
        iidx = lax.iota(jnp.int32, 16)

        for sb in range(SB):
            pltpu.async_copy(srows_v.at[pl.ds(sb * 16, 16)],
                             acc_sh.at[iidx], s_sems[sb], add=True)

        def chunk(o, b, refire_o):
            sb = b % SB
            isrc = src_v[pl.ds(o, 16)]
            idst = dst_v[pl.ds(o, 16)]
            pltpu.make_async_copy(h_hbm.at[isrc],
                                  grows_v.at[pl.ds(b * 16, 16)],
                                  g_sems[b]).wait()
            sd = plsc.load_gather(s1_v, [idst])
            ss = plsc.load_gather(s2_v, [isrc])
            t = sd + ss
            e = jnp.where(t >= 0.0, t, 0.01 * t)
            p = jnp.exp(e - cvec)
            plsc.addupdate_scatter(den_v, [idst], p)
            pltpu.make_async_copy(srows_v.at[pl.ds(sb * 16, 16)],
                                  acc_sh.at[idst], s_sems[sb]).wait()
            for r in range(16):
                pr = jnp.broadcast_to(p[r], (16,))
                for j in range(8):
                    sl = pl.ds(j * 16, 16)
                    srows_v[sb * 16 + r, sl] = grows_v[b * 16 + r, sl] * pr
            pltpu.async_copy(srows_v.at[pl.ds(sb * 16, 16)],
                             acc_sh.at[idst], s_sems[sb], add=True)
            if refire_o is not None:
                nsrc = src_v[pl.ds(refire_o, 16)]
                pltpu.async_copy(h_hbm.at[nsrc],
                                 grows_v.at[pl.ds(b * 16, 16)], g_sems[b])

        def refill(rr, carry):
            pltpu.sync_copy(src_hbm.at[pl.ds(ebase + rr * IDXCAP, IDXCAP)],
                            src_v)
            pltpu.sync_copy(dst_hbm.at[pl.ds(ebase + rr * IDXCAP, IDXCAP)],
                            dst_v)
            for b in range(GB):
                isrc0 = src_v[pl.ds(b * 16, 16)]
                pltpu.async_copy(h_hbm.at[isrc0],
                                 grows_v.at[pl.ds(b * 16, 16)], g_sems[b])

            def group(g, carry2):
                off = g * (GB * 16)
                for b in range(GB):
                    chunk(off + b * 16, b, off + GB * 16 + b * 16)
                return carry2

            lax.fori_loop(0, RG - 1, group, 0)
            off = (RG - 1) * (GB * 16)
            for b in range(GB):
                chunk(off + b * 16, b, None)
            return carry

        lax.fori_loop(0, EPT // IDXCAP, refill, 0)

        for sb in range(SB):
            pltpu.make_async_copy(srows_v.at[pl.ds(sb * 16, 16)],
                                  acc_sh.at[iidx], s_sems[sb]).wait()

        plsc.subcore_barrier()
        pltpu.sync_copy(acc_sh.at[pl.ds(rbase, RPT), :],
                        numer_hbm.at[cid, pl.ds(rbase, RPT), :])
        pltpu.sync_copy(den_v, den_hbm.at[wid])

    def body_wrap(src_hbm, dst_hbm, s1_hbm, s2_hbm, c_hbm, h_hbm,
                  numer_hbm, den_hbm,
                  src_v, dst_v, s1_v, s2_v, den_v, c_v, grows_v, srows_v,
                  acc_sh, g0, g1, g2, g3, g4, s0, s1s, set_sem):
        body(src_hbm, dst_hbm, s1_hbm, s2_hbm, c_hbm, h_hbm,
             numer_hbm, den_hbm,
             src_v, dst_v, s1_v, s2_v, den_v, c_v, grows_v, srows_v, acc_sh,
             [g0, g1, g2, g3, g4], [s0, s1s], set_sem)

    run = pl.kernel(
        body_wrap,
        out_type=[
            jax.ShapeDtypeStruct((NC, N_PAD, D), jnp.float32),
            jax.ShapeDtypeStruct((NW, N_PAD), jnp.float32),
        ],
        mesh=mesh,
        scratch_types=[
            pltpu.VMEM((IDXCAP,), jnp.int32),
            pltpu.VMEM((IDXCAP,), jnp.int32),
            pltpu.VMEM((N,), jnp.float32),
            pltpu.VMEM((N,), jnp.float32),
            pltpu.VMEM((N_PAD,), jnp.float32),
            pltpu.VMEM((16,), jnp.float32),
            pltpu.VMEM((GB * 16, D), jnp.float32),
            pltpu.VMEM((SB * 16, D), jnp.float32),
            pltpu.VMEM_SHARED((N_PAD, D), jnp.float32),
            pltpu.SemaphoreType.DMA,
            pltpu.SemaphoreType.DMA,
            pltpu.SemaphoreType.DMA,
            pltpu.SemaphoreType.DMA,
            pltpu.SemaphoreType.DMA,
            pltpu.SemaphoreType.DMA,
            pltpu.SemaphoreType.DMA,
            pltpu.SemaphoreType.DMA,
        ],
        compiler_params=pltpu.CompilerParams(needs_layout_passes=False),
    )
    return run(src, dst, s1, s2, c16, h)


def _sc_fin_body(numer_hbm, den_hbm, out_hbm, a_v, b_v, d_v, ds_v, sem):
    cid = lax.axis_index("c")
    sid = lax.axis_index("s")
    wid = cid * NS + sid
    base = wid * NPT

    pltpu.async_copy(numer_hbm.at[0, pl.ds(base, NPT), :], a_v, sem)
    pltpu.async_copy(numer_hbm.at[1, pl.ds(base, NPT), :], b_v, sem)
    for r in range(NW):
        pltpu.async_copy(den_hbm.at[pl.ds(r * N_PAD + base, NPT)],
                         d_v.at[pl.ds(r * NPT, NPT)], sem)
    pltpu.make_async_copy(numer_hbm.at[0, pl.ds(base, NPT), :], a_v,
                          sem).wait()
    pltpu.make_async_copy(numer_hbm.at[1, pl.ds(base, NPT), :], b_v,
                          sem).wait()
    for r in range(NW):
        pltpu.make_async_copy(den_hbm.at[pl.ds(r * N_PAD + base, NPT)],
                              d_v.at[pl.ds(r * NPT, NPT)], sem).wait()

    def dred(i, carry):
        sl = pl.ds(i * 16, 16)
        acc = d_v[pl.ds(i * 16, 16)]
        for r in range(1, NW):
            acc = acc + d_v[pl.ds(r * NPT + i * 16, 16)]
        ds_v[sl] = acc
        return carry

    lax.fori_loop(0, NPT // 16, dred, 0)

    def rowdiv(i, carry):
        inv = 1.0 / (ds_v[pl.ds(i * 16, 16)] + 1e-16)
        for rr in range(16):
            r = i * 16 + rr
            iv = jnp.broadcast_to(inv[rr], (16,))
            for j in range(8):
                sl = pl.ds(j * 16, 16)
                a_v[r, sl] = (a_v[r, sl] + b_v[r, sl]) * iv
        return carry

    lax.fori_loop(0, NPT // 16, rowdiv, 0)
    pltpu.sync_copy(a_v, out_hbm.at[pl.ds(base, NPT), :])


def _finalize(numer, denom):
    mesh = plsc.VectorSubcoreMesh(core_axis_name="c", subcore_axis_name="s",
                                  num_cores=NC, num_subcores=NS)
    run = pl.kernel(
        _sc_fin_body,
        out_type=jax.ShapeDtypeStruct((N_PAD, D), jnp.float32),
        mesh=mesh,
        scratch_types=[
            pltpu.VMEM((NPT, D), jnp.float32),
            pltpu.VMEM((NPT, D), jnp.float32),
            pltpu.VMEM((NW * NPT,), jnp.float32),
            pltpu.VMEM((NPT,), jnp.float32),
            pltpu.SemaphoreType.DMA,
        ],
    )
    return run(numer, denom)


def kernel(x, edge_index, W, b_w, a_w, a_b):
    a1 = a_w[:, :D]
    a2 = a_w[:, D:]
    h, s1, s2, c = _project(x, W, b_w.reshape(1, D), a1, a2, a_b)
    src = edge_index[0]
    dst = edge_index[1]
    c16 = jnp.broadcast_to(c, (16,))
    numer, denom = _edge_sweep(src, dst, s1.reshape(N), s2.reshape(N), c16, h)
    out = _finalize(numer, denom.reshape(NW * N_PAD))
    return out[:N]


# speedup vs baseline: 38.0472x; 1.0481x over previous
import jax
import jax.numpy as jnp
from jax import lax
from jax.experimental import pallas as pl
from jax.experimental.pallas import tpu as pltpu
from jax.experimental.pallas import tpu_sc as plsc

N = 10000
E = 320000
D = 128
N_PAD = 10240
NC = 2
NS = 16
NW = NC * NS
EPT = E // NW
RPT = N_PAD // NS
NPT = N_PAD // NW
BLK = 1000
GB = 5
SB = 2
IDXCAP = 2000


def _proj_body(x_ref, w_ref, b_ref, a1_ref, a2_ref, ab_ref,
               h_ref, s1_ref, s2_ref, c_ref, mx_ref):
    i = pl.program_id(0)
    x = x_ref[...]
    h = lax.dot_general(x, w_ref[...], (((1,), (1,)), ((), ())),
                        preferred_element_type=jnp.float32) + b_ref[...]
    h_ref[...] = h
    s1 = jnp.sum(h * a1_ref[...], axis=1, keepdims=True) + ab_ref[0]
    s2 = jnp.sum(h * a2_ref[...], axis=1, keepdims=True)
    s1_ref[...] = s1
    s2_ref[...] = s2

    @pl.when(i == 0)
    def _():
        mx_ref[0] = jnp.float32(-1e30)
        mx_ref[1] = jnp.float32(-1e30)

    mx_ref[0] = jnp.maximum(mx_ref[0], jnp.max(s1))
    mx_ref[1] = jnp.maximum(mx_ref[1], jnp.max(s2))

    @pl.when(i == (N // BLK) - 1)
    def _():
        t = mx_ref[0] + mx_ref[1]
        c_ref[0] = jnp.where(t >= 0.0, t, 0.01 * t)


def _project(x, W, b_w, a1, a2, a_b):
    return pl.pallas_call(
        _proj_body,
        grid=(N // BLK,),
        in_specs=[
            pl.BlockSpec((BLK, D), lambda i: (i, 0)),
            pl.BlockSpec((D, D), lambda i: (0, 0)),
            pl.BlockSpec((1, D), lambda i: (0, 0)),
            pl.BlockSpec((1, D), lambda i: (0, 0)),
            pl.BlockSpec((1, D), lambda i: (0, 0)),
            pl.BlockSpec(memory_space=pltpu.SMEM),
        ],
        out_specs=[
            pl.BlockSpec((BLK, D), lambda i: (i, 0)),
            pl.BlockSpec((BLK, 1), lambda i: (i, 0)),
            pl.BlockSpec((BLK, 1), lambda i: (i, 0)),
            pl.BlockSpec(memory_space=pltpu.SMEM),
        ],
        out_shape=[
            jax.ShapeDtypeStruct((N, D), jnp.float32),
            jax.ShapeDtypeStruct((N, 1), jnp.float32),
            jax.ShapeDtypeStruct((N, 1), jnp.float32),
            jax.ShapeDtypeStruct((1,), jnp.float32),
        ],
        scratch_shapes=[pltpu.SMEM((2,), jnp.float32)],
    )(x, W, b_w, a1, a2, a_b)


def _edge_sweep(src, dst, s1, s2, c16, h):
    mesh = plsc.VectorSubcoreMesh(core_axis_name="c", subcore_axis_name="s",
                                  num_cores=NC, num_subcores=NS)

    RG = IDXCAP // (GB * 16)

    def body(src_hbm, dst_hbm, s1_hbm, s2_hbm, c_hbm, h_hbm,
             numer_hbm, den_hbm,
             src_v, dst_v, s1_v, s2_v, den_v, c_v, grows_v, srows_v, acc_sh,
             g_sems, s_sems, set_sem):
        cid = lax.axis_index("c")
        sid = lax.axis_index("s")
        wid = cid * NS + sid
        ebase = wid * EPT
        rbase = sid * RPT

        pltpu.async_copy(s1_hbm, s1_v, set_sem)
        pltpu.async_copy(s2_hbm, s2_v, set_sem)
        pltpu.async_copy(c_hbm, c_v, set_sem)
        z16 = jnp.zeros((16,), jnp.float32)

        def zg(i, carry):
            for j in range(8):
                grows_v[i, pl.ds(j * 16, 16)] = z16
            return carry

        lax.fori_loop(0, GB * 16, zg, 0)

        def zs(i, carry):
            for j in range(8):
                srows_v[i, pl.ds(j * 16, 16)] = z16
            return carry

        lax.fori_loop(0, SB * 16, zs, 0)

        def zd(i, carry):
            den_v[pl.ds(i * 16, 16)] = z16
            return carry

        lax.fori_loop(0, N_PAD // 16, zd, 0)

        def za(i, carry):
            pltpu.async_copy(
                grows_v, acc_sh.at[pl.ds(rbase + i * (GB * 16), GB * 16), :],
                set_sem)
            return carry

        lax.fori_loop(0, RPT // (GB * 16), za, 0)

        pltpu.make_async_copy(s1_hbm, s1_v, set_sem).wait()
        pltpu.make_async_copy(s2_hbm, s2_v, set_sem).wait()
        pltpu.make_async_copy(c_hbm, c_v, set_sem).wait()

        def zaw(i, carry):
            pltpu.make_async_copy(
                grows_v, acc_sh.at[pl.ds(rbase + i * (GB * 16), GB * 16), :],
                set_sem).wait()
            return carry

        lax.fori_loop(0, RPT // (GB * 16), zaw, 0)
        plsc.subcore_barrier()
        cvec = c_v[...]
        iidx = lax.iota(jnp.int32, 16)

        for sb in range(SB):
            pltpu.async_copy(srows_v.at[pl.ds(sb * 16, 16)],
                             acc_sh.at[iidx], s_sems[sb], add=True)

        def chunk(o, b, refire_o):
            sb = b % SB
            isrc = src_v[pl.ds(o, 16)]
            idst = dst_v[pl.ds(o, 16)]
            pltpu.make_async_copy(h_hbm.at[isrc],
                                  grows_v.at[pl.ds(b * 16, 16)],
                                  g_sems[b]).wait()
            sd = plsc.load_gather(s1_v, [idst])
            ss = plsc.load_gather(s2_v, [isrc])
            t = sd + ss
            e = jnp.where(t >= 0.0, t, 0.01 * t)
            p = jnp.exp(e - cvec)
            plsc.addupdate_scatter(den_v, [idst], p)
            pltpu.make_async_copy(srows_v.at[pl.ds(sb * 16, 16)],
                                  acc_sh.at[idst], s_sems[sb]).wait()
            for r in range(16):
                pr = jnp.broadcast_to(p[r], (16,))
                for j in range(8):
                    sl = pl.ds(j * 16, 16)
                    srows_v[sb * 16 + r, sl] = grows_v[b * 16 + r, sl] * pr
            pltpu.async_copy(srows_v.at[pl.ds(sb * 16, 16)],
                             acc_sh.at[idst], s_sems[sb], add=True)
            if refire_o is not None:
                nsrc = src_v[pl.ds(refire_o, 16)]
                pltpu.async_copy(h_hbm.at[nsrc],
                                 grows_v.at[pl.ds(b * 16, 16)], g_sems[b])

        def refill(rr, carry):
            pltpu.sync_copy(src_hbm.at[pl.ds(ebase + rr * IDXCAP, IDXCAP)],
                            src_v)
            pltpu.sync_copy(dst_hbm.at[pl.ds(ebase + rr * IDXCAP, IDXCAP)],
                            dst_v)
            for b in range(GB):
                isrc0 = src_v[pl.ds(b * 16, 16)]
                pltpu.async_copy(h_hbm.at[isrc0],
                                 grows_v.at[pl.ds(b * 16, 16)], g_sems[b])

            def group(g, carry2):
                off = g * (GB * 16)
                for b in range(GB):
                    chunk(off + b * 16, b, off + GB * 16 + b * 16)
                return carry2

            lax.fori_loop(0, RG - 1, group, 0)
            off = (RG - 1) * (GB * 16)
            for b in range(GB):
                chunk(off + b * 16, b, None)
            return carry

        lax.fori_loop(0, EPT // IDXCAP, refill, 0)

        for sb in range(SB):
            pltpu.make_async_copy(srows_v.at[pl.ds(sb * 16, 16)],
                                  acc_sh.at[iidx], s_sems[sb]).wait()

        plsc.subcore_barrier()
        pltpu.sync_copy(acc_sh.at[pl.ds(rbase, RPT), :],
                        numer_hbm.at[cid, pl.ds(rbase, RPT), :])
        pltpu.sync_copy(den_v, den_hbm.at[wid])

    def body_wrap(src_hbm, dst_hbm, s1_hbm, s2_hbm, c_hbm, h_hbm,
                  numer_hbm, den_hbm,
                  src_v, dst_v, s1_v, s2_v, den_v, c_v, grows_v, srows_v,
                  acc_sh, g0, g1, g2, g3, g4, s0, s1s, set_sem):
        body(src_hbm, dst_hbm, s1_hbm, s2_hbm, c_hbm, h_hbm,
             numer_hbm, den_hbm,
             src_v, dst_v, s1_v, s2_v, den_v, c_v, grows_v, srows_v, acc_sh,
             [g0, g1, g2, g3, g4], [s0, s1s], set_sem)

    run = pl.kernel(
        body_wrap,
        out_type=[
            jax.ShapeDtypeStruct((NC, N_PAD, D), jnp.float32),
            jax.ShapeDtypeStruct((NW, N_PAD), jnp.float32),
        ],
        mesh=mesh,
        scratch_types=[
            pltpu.VMEM((IDXCAP,), jnp.int32),
            pltpu.VMEM((IDXCAP,), jnp.int32),
            pltpu.VMEM((N,), jnp.float32),
            pltpu.VMEM((N,), jnp.float32),
            pltpu.VMEM((N_PAD,), jnp.float32),
            pltpu.VMEM((16,), jnp.float32),
            pltpu.VMEM((GB * 16, D), jnp.float32),
            pltpu.VMEM((SB * 16, D), jnp.float32),
            pltpu.VMEM_SHARED((N_PAD, D), jnp.float32),
            pltpu.SemaphoreType.DMA,
            pltpu.SemaphoreType.DMA,
            pltpu.SemaphoreType.DMA,
            pltpu.SemaphoreType.DMA,
            pltpu.SemaphoreType.DMA,
            pltpu.SemaphoreType.DMA,
            pltpu.SemaphoreType.DMA,
            pltpu.SemaphoreType.DMA,
        ],
        compiler_params=pltpu.CompilerParams(needs_layout_passes=False),
    )
    return run(src, dst, s1, s2, c16, h)


TBLK = 1024


def _fin_body(numer_ref, den_ref, out_ref):
    n = numer_ref[0] + numer_ref[1]
    ones = jnp.ones((NW, 1), jnp.float32)
    dcol = lax.dot_general(den_ref[...], ones, (((0,), (0,)), ((), ())),
                           preferred_element_type=jnp.float32)
    recip = 1.0 / (dcol + 1e-16)
    out_ref[...] = n * jnp.broadcast_to(recip, (TBLK, D))


def _finalize(numer, denom):
    return pl.pallas_call(
        _fin_body,
        grid=(N_PAD // TBLK,),
        in_specs=[
            pl.BlockSpec((NC, TBLK, D), lambda i: (0, i, 0)),
            pl.BlockSpec((NW, TBLK), lambda i: (0, i)),
        ],
        out_specs=pl.BlockSpec((TBLK, D), lambda i: (i, 0)),
        out_shape=jax.ShapeDtypeStruct((N_PAD, D), jnp.float32),
    )(numer, denom)


def kernel(x, edge_index, W, b_w, a_w, a_b):
    a1 = a_w[:, :D]
    a2 = a_w[:, D:]
    h, s1, s2, c = _project(x, W, b_w.reshape(1, D), a1, a2, a_b)
    src = edge_index[0]
    dst = edge_index[1]
    c16 = jnp.broadcast_to(c, (16,))
    numer, denom = _edge_sweep(src, dst, s1.reshape(N), s2.reshape(N), c16, h)
    out = _finalize(numer, denom)
    return out[:N]


# speedup vs baseline: 38.1008x; 1.0014x over previous
import jax
import jax.numpy as jnp
from jax import lax
from jax.experimental import pallas as pl
from jax.experimental.pallas import tpu as pltpu
from jax.experimental.pallas import tpu_sc as plsc

N = 10000
E = 320000
D = 128
N_PAD = 10240
NC = 2
NS = 16
NW = NC * NS
EPT = E // NW
RPT = N_PAD // NS
NPT = N_PAD // NW
BLK = 1000
GB = 5
SB = 3
IDXCAP = 2000


def _proj_body(x_ref, w_ref, b_ref, a1_ref, a2_ref, ab_ref,
               h_ref, s1_ref, s2_ref, c_ref, mx_ref):
    i = pl.program_id(0)
    x = x_ref[...]
    h = lax.dot_general(x, w_ref[...], (((1,), (1,)), ((), ())),
                        preferred_element_type=jnp.float32) + b_ref[...]
    h_ref[...] = h
    s1 = jnp.sum(h * a1_ref[...], axis=1, keepdims=True) + ab_ref[0]
    s2 = jnp.sum(h * a2_ref[...], axis=1, keepdims=True)
    s1_ref[...] = s1
    s2_ref[...] = s2

    @pl.when(i == 0)
    def _():
        mx_ref[0] = jnp.float32(-1e30)
        mx_ref[1] = jnp.float32(-1e30)

    mx_ref[0] = jnp.maximum(mx_ref[0], jnp.max(s1))
    mx_ref[1] = jnp.maximum(mx_ref[1], jnp.max(s2))

    @pl.when(i == (N // BLK) - 1)
    def _():
        t = mx_ref[0] + mx_ref[1]
        c_ref[0] = jnp.where(t >= 0.0, t, 0.01 * t)


def _project(x, W, b_w, a1, a2, a_b):
    return pl.pallas_call(
        _proj_body,
        grid=(N // BLK,),
        in_specs=[
            pl.BlockSpec((BLK, D), lambda i: (i, 0)),
            pl.BlockSpec((D, D), lambda i: (0, 0)),
            pl.BlockSpec((1, D), lambda i: (0, 0)),
            pl.BlockSpec((1, D), lambda i: (0, 0)),
            pl.BlockSpec((1, D), lambda i: (0, 0)),
            pl.BlockSpec(memory_space=pltpu.SMEM),
        ],
        out_specs=[
            pl.BlockSpec((BLK, D), lambda i: (i, 0)),
            pl.BlockSpec((BLK, 1), lambda i: (i, 0)),
            pl.BlockSpec((BLK, 1), lambda i: (i, 0)),
            pl.BlockSpec(memory_space=pltpu.SMEM),
        ],
        out_shape=[
            jax.ShapeDtypeStruct((N, D), jnp.float32),
            jax.ShapeDtypeStruct((N, 1), jnp.float32),
            jax.ShapeDtypeStruct((N, 1), jnp.float32),
            jax.ShapeDtypeStruct((1,), jnp.float32),
        ],
        scratch_shapes=[pltpu.SMEM((2,), jnp.float32)],
    )(x, W, b_w, a1, a2, a_b)


def _edge_sweep(src, dst, s1, s2, c16, h):
    mesh = plsc.VectorSubcoreMesh(core_axis_name="c", subcore_axis_name="s",
                                  num_cores=NC, num_subcores=NS)

    RG = IDXCAP // (GB * 16)

    def body(src_hbm, dst_hbm, s1_hbm, s2_hbm, c_hbm, h_hbm,
             numer_hbm, den_hbm,
             src_v, dst_v, s1_v, s2_v, den_v, c_v, grows_v, srows_v, acc_sh,
             g_sems, s_sems, set_sem):
        cid = lax.axis_index("c")
        sid = lax.axis_index("s")
        wid = cid * NS + sid
        ebase = wid * EPT
        rbase = sid * RPT

        pltpu.async_copy(s1_hbm, s1_v, set_sem)
        pltpu.async_copy(s2_hbm, s2_v, set_sem)
        pltpu.async_copy(c_hbm, c_v, set_sem)
        z16 = jnp.zeros((16,), jnp.float32)

        def zg(i, carry):
            for j in range(8):
                grows_v[i, pl.ds(j * 16, 16)] = z16
            return carry

        lax.fori_loop(0, GB * 16, zg, 0)

        def zs(i, carry):
            for j in range(8):
                srows_v[i, pl.ds(j * 16, 16)] = z16
            return carry

        lax.fori_loop(0, SB * 16, zs, 0)

        def zd(i, carry):
            den_v[pl.ds(i * 16, 16)] = z16
            return carry

        lax.fori_loop(0, N_PAD // 16, zd, 0)

        def za(i, carry):
            pltpu.async_copy(
                grows_v, acc_sh.at[pl.ds(rbase + i * (GB * 16), GB * 16), :],
                set_sem)
            return carry

        def zaw(i, carry):
            pltpu.make_async_copy(
                grows_v, acc_sh.at[pl.ds(rbase + i * (GB * 16), GB * 16), :],
                set_sem).wait()
            return carry

        nz_full = RPT // (GB * 16)
        nz_last = (N - (NS - 1) * RPT) // (GB * 16)

        @pl.when(sid < NS - 1)
        def _():
            lax.fori_loop(0, nz_full, za, 0)

        @pl.when(sid == NS - 1)
        def _():
            lax.fori_loop(0, nz_last, za, 0)

        pltpu.make_async_copy(s1_hbm, s1_v, set_sem).wait()
        pltpu.make_async_copy(s2_hbm, s2_v, set_sem).wait()
        pltpu.make_async_copy(c_hbm, c_v, set_sem).wait()

        @pl.when(sid < NS - 1)
        def _():
            lax.fori_loop(0, nz_full, zaw, 0)

        @pl.when(sid == NS - 1)
        def _():
            lax.fori_loop(0, nz_last, zaw, 0)

        plsc.subcore_barrier()
        cvec = c_v[...]
        iidx = lax.iota(jnp.int32, 16)

        for sb in range(SB):
            pltpu.async_copy(srows_v.at[pl.ds(sb * 16, 16)],
                             acc_sh.at[iidx], s_sems[sb], add=True)

        def chunk(o, b, refire_o):
            sb = b % SB
            isrc = src_v[pl.ds(o, 16)]
            idst = dst_v[pl.ds(o, 16)]
            pltpu.make_async_copy(h_hbm.at[isrc],
                                  grows_v.at[pl.ds(b * 16, 16)],
                                  g_sems[b]).wait()
            sd = plsc.load_gather(s1_v, [idst])
            ss = plsc.load_gather(s2_v, [isrc])
            t = sd + ss
            e = jnp.where(t >= 0.0, t, 0.01 * t)
            p = jnp.exp(e - cvec)
            plsc.addupdate_scatter(den_v, [idst], p)
            pltpu.make_async_copy(srows_v.at[pl.ds(sb * 16, 16)],
                                  acc_sh.at[idst], s_sems[sb]).wait()
            for r in range(16):
                pr = jnp.broadcast_to(p[r], (16,))
                for j in range(8):
                    sl = pl.ds(j * 16, 16)
                    srows_v[sb * 16 + r, sl] = grows_v[b * 16 + r, sl] * pr
            pltpu.async_copy(srows_v.at[pl.ds(sb * 16, 16)],
                             acc_sh.at[idst], s_sems[sb], add=True)
            if refire_o is not None:
                nsrc = src_v[pl.ds(refire_o, 16)]
                pltpu.async_copy(h_hbm.at[nsrc],
                                 grows_v.at[pl.ds(b * 16, 16)], g_sems[b])

        def refill(rr, carry):
            pltpu.sync_copy(src_hbm.at[pl.ds(ebase + rr * IDXCAP, IDXCAP)],
                            src_v)
            pltpu.sync_copy(dst_hbm.at[pl.ds(ebase + rr * IDXCAP, IDXCAP)],
                            dst_v)
            for b in range(GB):
                isrc0 = src_v[pl.ds(b * 16, 16)]
                pltpu.async_copy(h_hbm.at[isrc0],
                                 grows_v.at[pl.ds(b * 16, 16)], g_sems[b])

            def group(g, carry2):
                off = g * (GB * 16)
                for b in range(GB):
                    chunk(off + b * 16, b, off + GB * 16 + b * 16)
                return carry2

            lax.fori_loop(0, RG - 1, group, 0)
            off = (RG - 1) * (GB * 16)
            for b in range(GB):
                chunk(off + b * 16, b, None)
            return carry

        lax.fori_loop(0, EPT // IDXCAP, refill, 0)

        for sb in range(SB):
            pltpu.make_async_copy(srows_v.at[pl.ds(sb * 16, 16)],
                                  acc_sh.at[iidx], s_sems[sb]).wait()

        plsc.subcore_barrier()

        @pl.when(sid < NS - 1)
        def _():
            pltpu.sync_copy(acc_sh.at[pl.ds(rbase, RPT), :],
                            numer_hbm.at[cid, pl.ds(rbase, RPT), :])

        @pl.when(sid == NS - 1)
        def _():
            last = N - (NS - 1) * RPT
            pltpu.sync_copy(acc_sh.at[pl.ds((NS - 1) * RPT, last), :],
                            numer_hbm.at[cid, pl.ds((NS - 1) * RPT, last), :])

        pltpu.sync_copy(den_v, den_hbm.at[wid])

    def body_wrap(src_hbm, dst_hbm, s1_hbm, s2_hbm, c_hbm, h_hbm,
                  numer_hbm, den_hbm,
                  src_v, dst_v, s1_v, s2_v, den_v, c_v, grows_v, srows_v,
                  acc_sh, g0, g1, g2, g3, g4, s0, s1s, s2s, set_sem):
        body(src_hbm, dst_hbm, s1_hbm, s2_hbm, c_hbm, h_hbm,
             numer_hbm, den_hbm,
             src_v, dst_v, s1_v, s2_v, den_v, c_v, grows_v, srows_v, acc_sh,
             [g0, g1, g2, g3, g4], [s0, s1s, s2s], set_sem)

    run = pl.kernel(
        body_wrap,
        out_type=[
            jax.ShapeDtypeStruct((NC, N_PAD, D), jnp.float32),
            jax.ShapeDtypeStruct((NW, N_PAD), jnp.float32),
        ],
        mesh=mesh,
        scratch_types=[
            pltpu.VMEM((IDXCAP,), jnp.int32),
            pltpu.VMEM((IDXCAP,), jnp.int32),
            pltpu.VMEM((N,), jnp.float32),
            pltpu.VMEM((N,), jnp.float32),
            pltpu.VMEM((N_PAD,), jnp.float32),
            pltpu.VMEM((16,), jnp.float32),
            pltpu.VMEM((GB * 16, D), jnp.float32),
            pltpu.VMEM((SB * 16, D), jnp.float32),
            pltpu.VMEM_SHARED((N, D), jnp.float32),
            pltpu.SemaphoreType.DMA,
            pltpu.SemaphoreType.DMA,
            pltpu.SemaphoreType.DMA,
            pltpu.SemaphoreType.DMA,
            pltpu.SemaphoreType.DMA,
            pltpu.SemaphoreType.DMA,
            pltpu.SemaphoreType.DMA,
            pltpu.SemaphoreType.DMA,
            pltpu.SemaphoreType.DMA,
        ],
        compiler_params=pltpu.CompilerParams(needs_layout_passes=False),
    )
    return run(src, dst, s1, s2, c16, h)


TBLK = 1024


def _fin_body(numer_ref, den_ref, out_ref):
    n = numer_ref[0] + numer_ref[1]
    ones = jnp.ones((NW, 1), jnp.float32)
    dcol = lax.dot_general(den_ref[...], ones, (((0,), (0,)), ((), ())),
                           preferred_element_type=jnp.float32)
    recip = 1.0 / (dcol + 1e-16)
    out_ref[...] = n * jnp.broadcast_to(recip, (TBLK, D))


def _finalize(numer, denom):
    return pl.pallas_call(
        _fin_body,
        grid=(N_PAD // TBLK,),
        in_specs=[
            pl.BlockSpec((NC, TBLK, D), lambda i: (0, i, 0)),
            pl.BlockSpec((NW, TBLK), lambda i: (0, i)),
        ],
        out_specs=pl.BlockSpec((TBLK, D), lambda i: (i, 0)),
        out_shape=jax.ShapeDtypeStruct((N_PAD, D), jnp.float32),
    )(numer, denom)


def kernel(x, edge_index, W, b_w, a_w, a_b):
    a1 = a_w[:, :D]
    a2 = a_w[:, D:]
    h, s1, s2, c = _project(x, W, b_w.reshape(1, D), a1, a2, a_b)
    src = edge_index[0]
    dst = edge_index[1]
    c16 = jnp.broadcast_to(c, (16,))
    numer, denom = _edge_sweep(src, dst, s1.reshape(N), s2.reshape(N), c16, h)
    out = _finalize(numer, denom)
    return out[:N]
